# baseline (speedup 1.0000x reference)
import functools

import jax
import jax.numpy as jnp
from jax import lax
from jax.experimental import pallas as pl
from jax.experimental.pallas import tpu as pltpu
from jax.experimental.pallas import tpu_sc as plsc

_LIKELIHOOD_TOPK = 5
_NC, _NS, _L = 2, 16, 16
_NW = _NC * _NS


def _sc_body(l_hbm, t_hbm, cnt_out, zs_out, tbuf, lbuf, stage, *, slice_len, chunk):
    wid = lax.axis_index("s") * _NC + lax.axis_index("c")
    base = wid * slice_len
    nchunk = slice_len // chunk
    steps = chunk // _L

    cnt = jnp.zeros((_L,), jnp.float32)
    zs = jnp.zeros((_L,), jnp.float32)
    for c in range(nchunk):
        pltpu.sync_copy(t_hbm.at[pl.ds(base + c * chunk, chunk)], tbuf)
        pltpu.sync_copy(l_hbm.at[pl.ds(base + c * chunk, chunk)], lbuf)

        def step(j, carry, tbuf=tbuf, lbuf=lbuf):
            cnt, zs = carry
            t = tbuf[pl.ds(j * _L, _L)]
            l = lbuf[pl.ds(j * _L, _L)]
            cnt = cnt + jnp.where(t != t, jnp.float32(1.0), jnp.float32(0.0))
            zs = zs + jnp.float32(0.0) * l
            return (cnt, zs)

        cnt, zs = lax.fori_loop(0, steps, step, (cnt, zs), unroll=4)

    stage[pl.ds(0, _L)] = cnt
    pltpu.sync_copy(stage.at[pl.ds(0, _L)], cnt_out.at[pl.ds(wid * _L, _L)])
    stage[pl.ds(_L, _L)] = zs
    pltpu.sync_copy(stage.at[pl.ds(_L, _L)], zs_out.at[pl.ds(wid * _L, _L)])


def _finalize_body(cnt_ref, zs_ref, out_ref, *, total, num_top_k):
    nan_cnt = jnp.sum(cnt_ref[...])
    loss_sum = jnp.sum(zs_ref[...])
    denom = jnp.float32(total) - jnp.minimum(nan_cnt, jnp.float32(num_top_k))
    out_ref[0, 0] = loss_sum / denom


def kernel(logits, targets):
    B, N = targets.shape
    total = B * N
    slice_len = total // _NW
    chunk = min(slice_len, 32768)

    mesh = plsc.VectorSubcoreMesh(core_axis_name="c", subcore_axis_name="s")
    sc_partials = functools.partial(
        pl.kernel,
        mesh=mesh,
        out_type=[
            jax.ShapeDtypeStruct((_NW * _L,), jnp.float32),
            jax.ShapeDtypeStruct((_NW * _L,), jnp.float32),
        ],
        scratch_types=[
            pltpu.VMEM((chunk,), jnp.float32),
            pltpu.VMEM((chunk,), jnp.float32),
            pltpu.VMEM((2 * _L,), jnp.float32),
        ],
    )(functools.partial(_sc_body, slice_len=slice_len, chunk=chunk))

    cnt, zs = sc_partials(logits.reshape(-1), targets.reshape(-1))

    out = pl.pallas_call(
        functools.partial(
            _finalize_body, total=float(total), num_top_k=float(_LIKELIHOOD_TOPK * B)
        ),
        out_specs=pl.BlockSpec(memory_space=pltpu.SMEM),
        out_shape=jax.ShapeDtypeStruct((1, 1), jnp.float32),
    )(cnt.reshape(8, _NW * _L // 8), zs.reshape(8, _NW * _L // 8))
    return out[0, 0]


# speedup vs baseline: 1.5822x; 1.5822x over previous
import functools

import jax
import jax.numpy as jnp
from jax import lax
from jax.experimental import pallas as pl
from jax.experimental.pallas import tpu as pltpu
from jax.experimental.pallas import tpu_sc as plsc

_LIKELIHOOD_TOPK = 5
_NC, _NS, _L = 2, 16, 16
_NW = _NC * _NS
_ACC = 4
_EXP_MASK = 0x7FFFFFFF
_INF_BITS = 0x7F800000


def _sc_count_body(t_hbm, cnt_out, buf0, buf1, stage, sem0, sem1, *, slice_len, chunk):
    wid = lax.axis_index("s") * _NC + lax.axis_index("c")
    base = wid * slice_len
    nchunk = slice_len // chunk
    bufs = (buf0, buf1)
    sems = (sem0, sem1)

    copy = pltpu.async_copy(t_hbm.at[pl.ds(base, chunk)], buf0, sem0)
    accs = tuple(jnp.zeros((_L,), jnp.int32) for _ in range(_ACC))
    for c in range(nchunk):
        if c + 1 < nchunk:
            nxt = pltpu.async_copy(
                t_hbm.at[pl.ds(base + (c + 1) * chunk, chunk)],
                bufs[(c + 1) % 2],
                sems[(c + 1) % 2],
            )
        copy.wait()
        buf = bufs[c % 2]

        def step(j, accs, buf=buf):
            out = []
            for k in range(_ACC):
                t = buf[pl.ds((j * _ACC + k) * _L, _L)]
                u = lax.bitcast_convert_type(t, jnp.int32)
                is_nan = (u & _EXP_MASK) > _INF_BITS
                out.append(accs[k] + jnp.where(is_nan, 1, 0))
            return tuple(out)

        accs = lax.fori_loop(0, chunk // (_L * _ACC), step, accs, unroll=2)
        if c + 1 < nchunk:
            copy = nxt

    cnt = (accs[0] + accs[1]) + (accs[2] + accs[3])
    stage[...] = cnt.astype(jnp.float32)
    pltpu.sync_copy(stage, cnt_out.at[pl.ds(wid * _L, _L)])


def _tc_zsum_body(l_ref, zs_ref, acc_ref):
    i = pl.program_id(0)

    @pl.when(i == 0)
    def _init():
        acc_ref[0] = jnp.float32(0.0)

    acc_ref[0] = acc_ref[0] + jnp.sum(jnp.float32(0.0) * l_ref[...]) + jnp.sum(
        jnp.float32(0.0) * (-l_ref[...])
    )

    @pl.when(i == pl.num_programs(0) - 1)
    def _emit():
        zs_ref[0] = acc_ref[0]


def _finalize_body(cnt_ref, zs_ref, out_ref, *, total, num_top_k):
    nan_cnt = jnp.sum(cnt_ref[...])
    denom = jnp.float32(total) - jnp.minimum(nan_cnt, jnp.float32(num_top_k))
    out_ref[0, 0] = zs_ref[0] / denom


def kernel(logits, targets):
    B, N = targets.shape
    total = B * N
    slice_len = total // _NW
    chunk = min(slice_len, 16384)

    mesh = plsc.VectorSubcoreMesh(core_axis_name="c", subcore_axis_name="s")
    sc_count = functools.partial(
        pl.kernel,
        mesh=mesh,
        out_type=jax.ShapeDtypeStruct((_NW * _L,), jnp.float32),
        scratch_types=[
            pltpu.VMEM((chunk,), jnp.float32),
            pltpu.VMEM((chunk,), jnp.float32),
            pltpu.VMEM((_L,), jnp.float32),
            pltpu.SemaphoreType.DMA,
            pltpu.SemaphoreType.DMA,
        ],
    )(functools.partial(_sc_count_body, slice_len=slice_len, chunk=chunk))

    cnt = sc_count(targets.reshape(-1))

    grid = 16
    zs = pl.pallas_call(
        _tc_zsum_body,
        grid=(grid,),
        in_specs=[pl.BlockSpec((B, N // grid), lambda i: (0, i))],
        out_specs=pl.BlockSpec(memory_space=pltpu.SMEM),
        out_shape=jax.ShapeDtypeStruct((1,), jnp.float32),
        scratch_shapes=[pltpu.SMEM((1,), jnp.float32)],
    )(logits)

    out = pl.pallas_call(
        functools.partial(
            _finalize_body, total=float(total), num_top_k=float(_LIKELIHOOD_TOPK * B)
        ),
        in_specs=[
            pl.BlockSpec((8, _NW * _L // 8), lambda: (0, 0)),
            pl.BlockSpec(memory_space=pltpu.SMEM),
        ],
        out_specs=pl.BlockSpec(memory_space=pltpu.SMEM),
        out_shape=jax.ShapeDtypeStruct((1, 1), jnp.float32),
    )(cnt.reshape(8, _NW * _L // 8), zs)
    return out[0, 0]


# speedup vs baseline: 2.3107x; 1.4604x over previous
import functools

import jax
import jax.numpy as jnp
from jax import lax
from jax.experimental import pallas as pl
from jax.experimental.pallas import tpu as pltpu
from jax.experimental.pallas import tpu_sc as plsc

_LIKELIHOOD_TOPK = 5
_NC, _NS, _L = 2, 16, 16
_NW = _NC * _NS
_ACC = 4
_EXP_MASK = 0x7FFFFFFF
_INF_BITS = 0x7F800000


def _sc_count_body(t_hbm, cnt_out, buf0, buf1, stage, sem0, sem1, *, rows_per_w, chunk):
    wid = lax.axis_index("s") * _NC + lax.axis_index("c")
    row0 = wid * rows_per_w
    ncols = t_hbm.shape[1]
    cpr = ncols // chunk
    nchunk = rows_per_w * cpr
    bufs = (buf0, buf1)
    sems = (sem0, sem1)

    def _slice(c):
        return t_hbm.at[row0 + c // cpr, pl.ds((c % cpr) * chunk, chunk)]

    copy = pltpu.async_copy(_slice(0), buf0, sem0)
    accs = tuple(jnp.zeros((_L,), jnp.int32) for _ in range(_ACC))
    for c in range(nchunk):
        if c + 1 < nchunk:
            nxt = pltpu.async_copy(_slice(c + 1), bufs[(c + 1) % 2], sems[(c + 1) % 2])
        copy.wait()
        buf = bufs[c % 2]

        def step(j, accs, buf=buf):
            out = []
            for k in range(_ACC):
                t = buf[pl.ds((j * _ACC + k) * _L, _L)]
                u = lax.bitcast_convert_type(t, jnp.int32)
                is_nan = (u & _EXP_MASK) > _INF_BITS
                out.append(accs[k] + jnp.where(is_nan, 1, 0))
            return tuple(out)

        accs = lax.fori_loop(0, chunk // (_L * _ACC), step, accs, unroll=2)
        if c + 1 < nchunk:
            copy = nxt

    cnt = (accs[0] + accs[1]) + (accs[2] + accs[3])
    stage[...] = cnt.astype(jnp.float32)
    pltpu.sync_copy(stage, cnt_out.at[pl.ds(wid * _L, _L)])


def _tc_zsum_body(l_ref, zs_ref, acc_ref):
    i = pl.program_id(0)

    @pl.when(i == 0)
    def _init():
        acc_ref[0] = jnp.float32(0.0)

    acc_ref[0] = acc_ref[0] + jnp.sum(jnp.float32(0.0) * l_ref[...]) + jnp.sum(
        jnp.float32(0.0) * (-l_ref[...])
    )

    @pl.when(i == pl.num_programs(0) - 1)
    def _emit():
        zs_ref[0] = acc_ref[0]


def _finalize_body(cnt_ref, zs_ref, out_ref, *, total, num_top_k):
    nan_cnt = jnp.sum(cnt_ref[...])
    denom = jnp.float32(total) - jnp.minimum(nan_cnt, jnp.float32(num_top_k))
    out_ref[0, 0] = zs_ref[0] / denom


def kernel(logits, targets):
    B, N = targets.shape
    total = B * N
    rows_per_w = B // _NW
    chunk = min(N, 16384)

    mesh = plsc.VectorSubcoreMesh(core_axis_name="c", subcore_axis_name="s")
    sc_count = functools.partial(
        pl.kernel,
        mesh=mesh,
        out_type=jax.ShapeDtypeStruct((_NW * _L,), jnp.float32),
        scratch_types=[
            pltpu.VMEM((chunk,), jnp.float32),
            pltpu.VMEM((chunk,), jnp.float32),
            pltpu.VMEM((_L,), jnp.float32),
            pltpu.SemaphoreType.DMA,
            pltpu.SemaphoreType.DMA,
        ],
    )(functools.partial(_sc_count_body, rows_per_w=rows_per_w, chunk=chunk))

    cnt = sc_count(targets)

    grid = 16
    zs = pl.pallas_call(
        _tc_zsum_body,
        grid=(grid,),
        in_specs=[pl.BlockSpec((B, N // grid), lambda i: (0, i))],
        out_specs=pl.BlockSpec(memory_space=pltpu.SMEM),
        out_shape=jax.ShapeDtypeStruct((1,), jnp.float32),
        scratch_shapes=[pltpu.SMEM((1,), jnp.float32)],
    )(logits)

    out = pl.pallas_call(
        functools.partial(
            _finalize_body, total=float(total), num_top_k=float(_LIKELIHOOD_TOPK * B)
        ),
        in_specs=[
            pl.BlockSpec((8, _NW * _L // 8), lambda: (0, 0)),
            pl.BlockSpec(memory_space=pltpu.SMEM),
        ],
        out_specs=pl.BlockSpec(memory_space=pltpu.SMEM),
        out_shape=jax.ShapeDtypeStruct((1, 1), jnp.float32),
    )(cnt.reshape(8, _NW * _L // 8), zs)
    return out[0, 0]


# speedup vs baseline: 2.4667x; 1.0675x over previous
import functools

import jax
import jax.numpy as jnp
from jax import lax
from jax.experimental import pallas as pl
from jax.experimental.pallas import tpu as pltpu
from jax.experimental.pallas import tpu_sc as plsc

_LIKELIHOOD_TOPK = 5
_NC, _NS, _L = 2, 16, 16
_NW = _NC * _NS
_ACC = 4
_EXP_MASK = 0x7FFFFFFF
_INF_BITS = 0x7F800000


def _sc_count_body(t_hbm, cnt_out, buf0, buf1, stage, sem0, sem1, *, rows_per_w, chunk):
    wid = lax.axis_index("s") * _NC + lax.axis_index("c")
    row0 = wid * rows_per_w
    ncols = t_hbm.shape[1]
    cpr = ncols // chunk
    nchunk = rows_per_w * cpr
    bufs = (buf0, buf1)
    sems = (sem0, sem1)

    def _slice(c):
        return t_hbm.at[row0 + c // cpr, pl.ds((c % cpr) * chunk, chunk)]

    copy = pltpu.async_copy(_slice(0), buf0, sem0)
    accs = tuple(jnp.zeros((_L,), jnp.int32) for _ in range(_ACC))
    for c in range(nchunk):
        if c + 1 < nchunk:
            nxt = pltpu.async_copy(_slice(c + 1), bufs[(c + 1) % 2], sems[(c + 1) % 2])
        copy.wait()
        buf = bufs[c % 2]

        def step(j, accs, buf=buf):
            out = []
            for k in range(_ACC):
                t = buf[pl.ds((j * _ACC + k) * _L, _L)]
                u = lax.bitcast_convert_type(t, jnp.int32)
                is_nan = (u & _EXP_MASK) > _INF_BITS
                out.append(accs[k] + jnp.where(is_nan, 1, 0))
            return tuple(out)

        accs = lax.fori_loop(0, chunk // (_L * _ACC), step, accs, unroll=2)
        if c + 1 < nchunk:
            copy = nxt

    cnt = (accs[0] + accs[1]) + (accs[2] + accs[3])
    stage[...] = cnt.astype(jnp.float32)
    pltpu.sync_copy(stage, cnt_out.at[pl.ds(wid * _L, _L)])


def _tc_zsum_body(la_ref, lb_ref, zs_ref, acc_ref):
    i = pl.program_id(0)

    @pl.when(i == 0)
    def _init():
        acc_ref[0] = jnp.float32(0.0)

    za = jnp.sum(jnp.float32(0.0) * la_ref[...]) + jnp.sum(
        jnp.float32(0.0) * (-la_ref[...])
    )
    zb = jnp.sum(jnp.float32(0.0) * lb_ref[...]) + jnp.sum(
        jnp.float32(0.0) * (-lb_ref[...])
    )
    acc_ref[0] = acc_ref[0] + za + zb

    @pl.when(i == pl.num_programs(0) - 1)
    def _emit():
        zs_ref[0] = acc_ref[0]


def _finalize_body(cnt_ref, zs_ref, out_ref, *, total, num_top_k):
    nan_cnt = jnp.sum(cnt_ref[...])
    denom = jnp.float32(total) - jnp.minimum(nan_cnt, jnp.float32(num_top_k))
    out_ref[0, 0] = zs_ref[0] / denom


def kernel(logits, targets):
    B, N = targets.shape
    total = B * N
    rows_per_w = B // _NW
    chunk = min(N, 32768)

    mesh = plsc.VectorSubcoreMesh(core_axis_name="c", subcore_axis_name="s")
    sc_count = functools.partial(
        pl.kernel,
        mesh=mesh,
        out_type=jax.ShapeDtypeStruct((_NW * _L,), jnp.float32),
        scratch_types=[
            pltpu.VMEM((chunk,), jnp.float32),
            pltpu.VMEM((chunk,), jnp.float32),
            pltpu.VMEM((_L,), jnp.float32),
            pltpu.SemaphoreType.DMA,
            pltpu.SemaphoreType.DMA,
        ],
    )(functools.partial(_sc_count_body, rows_per_w=rows_per_w, chunk=chunk))

    cnt = sc_count(targets)

    grid = 8
    blk = N // (2 * grid)
    zs = pl.pallas_call(
        _tc_zsum_body,
        grid=(grid,),
        in_specs=[
            pl.BlockSpec((B, blk), lambda i: (0, i)),
            pl.BlockSpec((B, blk), lambda i, g=grid: (0, i + g)),
        ],
        out_specs=pl.BlockSpec(memory_space=pltpu.SMEM),
        out_shape=jax.ShapeDtypeStruct((1,), jnp.float32),
        scratch_shapes=[pltpu.SMEM((1,), jnp.float32)],
    )(logits, logits)

    out = pl.pallas_call(
        functools.partial(
            _finalize_body, total=float(total), num_top_k=float(_LIKELIHOOD_TOPK * B)
        ),
        in_specs=[
            pl.BlockSpec((_NW * _L,), lambda: (0,)),
            pl.BlockSpec(memory_space=pltpu.SMEM),
        ],
        out_specs=pl.BlockSpec(memory_space=pltpu.SMEM),
        out_shape=jax.ShapeDtypeStruct((1, 1), jnp.float32),
    )(cnt, zs)
    return out[0, 0]


# speedup vs baseline: 2.4723x; 1.0023x over previous
import functools

import jax
import jax.numpy as jnp
from jax import lax
from jax.experimental import pallas as pl
from jax.experimental.pallas import tpu as pltpu
from jax.experimental.pallas import tpu_sc as plsc

_LIKELIHOOD_TOPK = 5
_NC, _NS, _L = 2, 16, 16
_NW = _NC * _NS
_ACC = 4
_EXP_MASK = 0x7FFFFFFF
_INF_BITS = 0x7F800000


def _sc_count_body(t_hbm, cnt_out, buf0, buf1, stage, sem0, sem1, *, chunk_cols):
    nrows, ncols = t_hbm.shape
    bands = nrows // 8
    halves = _NW // bands
    colspan = ncols // halves
    nchunk = colspan // chunk_cols

    wid = lax.axis_index("s") * _NC + lax.axis_index("c")
    band = wid // halves
    col0 = (wid % halves) * colspan
    bufs = (buf0, buf1)
    sems = (sem0, sem1)

    def _slice(c):
        return t_hbm.at[pl.ds(band * 8, 8), pl.ds(col0 + c * chunk_cols, chunk_cols)]

    copy = pltpu.async_copy(_slice(0), buf0, sem0)
    accs = tuple(jnp.zeros((_L,), jnp.int32) for _ in range(_ACC))
    for c in range(nchunk):
        if c + 1 < nchunk:
            nxt = pltpu.async_copy(_slice(c + 1), bufs[(c + 1) % 2], sems[(c + 1) % 2])
        copy.wait()
        buf = bufs[c % 2]

        def row_loop(r, accs, buf=buf):
            def step(j, accs, buf=buf, r=r):
                out = []
                for k in range(_ACC):
                    t = buf[r, pl.ds((j * _ACC + k) * _L, _L)]
                    u = lax.bitcast_convert_type(t, jnp.int32)
                    is_nan = (u & _EXP_MASK) > _INF_BITS
                    out.append(accs[k] + jnp.where(is_nan, 1, 0))
                return tuple(out)

            return lax.fori_loop(0, chunk_cols // (_L * _ACC), step, accs, unroll=2)

        accs = lax.fori_loop(0, 8, row_loop, accs)
        if c + 1 < nchunk:
            copy = nxt

    cnt = (accs[0] + accs[1]) + (accs[2] + accs[3])
    stage[...] = cnt.astype(jnp.float32)
    pltpu.sync_copy(stage, cnt_out.at[pl.ds(wid * _L, _L)])


def _tc_zsum_body(la_ref, lb_ref, zs_ref, acc_ref):
    i = pl.program_id(0)

    @pl.when(i == 0)
    def _init():
        acc_ref[0] = jnp.float32(0.0)

    za = jnp.sum(jnp.float32(0.0) * la_ref[...]) + jnp.sum(
        jnp.float32(0.0) * (-la_ref[...])
    )
    zb = jnp.sum(jnp.float32(0.0) * lb_ref[...]) + jnp.sum(
        jnp.float32(0.0) * (-lb_ref[...])
    )
    acc_ref[0] = acc_ref[0] + za + zb

    @pl.when(i == pl.num_programs(0) - 1)
    def _emit():
        zs_ref[0] = acc_ref[0]


def _finalize_body(cnt_ref, zs_ref, out_ref, *, total, num_top_k):
    nan_cnt = jnp.sum(cnt_ref[...])
    denom = jnp.float32(total) - jnp.minimum(nan_cnt, jnp.float32(num_top_k))
    out_ref[0, 0] = zs_ref[0] / denom


def kernel(logits, targets):
    B, N = targets.shape
    total = B * N
    chunk_cols = 4096

    mesh = plsc.VectorSubcoreMesh(core_axis_name="c", subcore_axis_name="s")
    sc_count = functools.partial(
        pl.kernel,
        mesh=mesh,
        out_type=jax.ShapeDtypeStruct((_NW * _L,), jnp.float32),
        scratch_types=[
            pltpu.VMEM((8, chunk_cols), jnp.float32),
            pltpu.VMEM((8, chunk_cols), jnp.float32),
            pltpu.VMEM((_L,), jnp.float32),
            pltpu.SemaphoreType.DMA,
            pltpu.SemaphoreType.DMA,
        ],
    )(functools.partial(_sc_count_body, chunk_cols=chunk_cols))

    cnt = sc_count(targets)

    grid = 8
    blk = N // (2 * grid)
    zs = pl.pallas_call(
        _tc_zsum_body,
        grid=(grid,),
        in_specs=[
            pl.BlockSpec((B, blk), lambda i: (0, i)),
            pl.BlockSpec((B, blk), lambda i, g=grid: (0, i + g)),
        ],
        out_specs=pl.BlockSpec(memory_space=pltpu.SMEM),
        out_shape=jax.ShapeDtypeStruct((1,), jnp.float32),
        scratch_shapes=[pltpu.SMEM((1,), jnp.float32)],
    )(logits, logits)

    out = pl.pallas_call(
        functools.partial(
            _finalize_body, total=float(total), num_top_k=float(_LIKELIHOOD_TOPK * B)
        ),
        in_specs=[
            pl.BlockSpec((_NW * _L,), lambda: (0,)),
            pl.BlockSpec(memory_space=pltpu.SMEM),
        ],
        out_specs=pl.BlockSpec(memory_space=pltpu.SMEM),
        out_shape=jax.ShapeDtypeStruct((1, 1), jnp.float32),
    )(cnt, zs)
    return out[0, 0]


# speedup vs baseline: 2.5809x; 1.0439x over previous
import functools

import jax
import jax.numpy as jnp
from jax import lax
from jax.experimental import pallas as pl
from jax.experimental.pallas import tpu as pltpu
from jax.experimental.pallas import tpu_sc as plsc

_LIKELIHOOD_TOPK = 5
_NC, _NS, _L = 2, 16, 16
_NW = _NC * _NS
_ACC = 4
_EXP_MASK = 0x7FFFFFFF
_INF_BITS = 0x7F800000


def _sc_count_body(t_hbm, cnt_out, buf0, buf1, stage, sem0, sem1, *, sc_cols, chunk_cols):
    nrows, _ = t_hbm.shape
    bands = nrows // 8
    halves = _NW // bands
    colspan = sc_cols // halves
    nchunk = colspan // chunk_cols

    wid = lax.axis_index("s") * _NC + lax.axis_index("c")
    band = wid // halves
    col0 = (wid % halves) * colspan
    bufs = (buf0, buf1)
    sems = (sem0, sem1)

    def _slice(c):
        return t_hbm.at[pl.ds(band * 8, 8), pl.ds(col0 + c * chunk_cols, chunk_cols)]

    copy = pltpu.async_copy(_slice(0), buf0, sem0)
    accs = tuple(jnp.zeros((_L,), jnp.int32) for _ in range(_ACC))
    for c in range(nchunk):
        if c + 1 < nchunk:
            nxt = pltpu.async_copy(_slice(c + 1), bufs[(c + 1) % 2], sems[(c + 1) % 2])
        copy.wait()
        buf = bufs[c % 2]

        def row_loop(r, accs, buf=buf):
            def step(j, accs, buf=buf, r=r):
                out = []
                for k in range(_ACC):
                    t = buf[r, pl.ds((j * _ACC + k) * _L, _L)]
                    u = lax.bitcast_convert_type(t, jnp.int32)
                    is_nan = (u & _EXP_MASK) > _INF_BITS
                    out.append(accs[k] + jnp.where(is_nan, 1, 0))
                return tuple(out)

            return lax.fori_loop(0, chunk_cols // (_L * _ACC), step, accs, unroll=2)

        accs = lax.fori_loop(0, 8, row_loop, accs)
        if c + 1 < nchunk:
            copy = nxt

    cnt = (accs[0] + accs[1]) + (accs[2] + accs[3])
    stage[...] = cnt.astype(jnp.float32)
    pltpu.sync_copy(stage, cnt_out.at[pl.ds(wid * _L, _L)])


def _tc_dense_body(l0, l1, l2, l3, tr, zs_ref, acc_ref):
    i = pl.program_id(0)

    @pl.when(i == 0)
    def _init():
        acc_ref[0] = jnp.float32(0.0)
        acc_ref[1] = jnp.float32(0.0)

    z = jnp.float32(0.0)
    for l_ref in (l0, l1, l2, l3):
        l = l_ref[...]
        z = z + jnp.sum(jnp.float32(0.0) * l) + jnp.sum(jnp.float32(0.0) * (-l))
    u = lax.bitcast_convert_type(tr[...], jnp.int32)
    is_nan = (u & _EXP_MASK) > _INF_BITS
    c = jnp.sum(jnp.where(is_nan, 1, 0)).astype(jnp.float32)
    acc_ref[0] = acc_ref[0] + z
    acc_ref[1] = acc_ref[1] + c

    @pl.when(i == pl.num_programs(0) - 1)
    def _emit():
        zs_ref[0] = acc_ref[0]
        zs_ref[1] = acc_ref[1]


def _finalize_body(cnt_ref, zs_ref, out_ref, *, total, num_top_k):
    nan_cnt = jnp.sum(cnt_ref[...]) + zs_ref[1]
    denom = jnp.float32(total) - jnp.minimum(nan_cnt, jnp.float32(num_top_k))
    out_ref[0, 0] = zs_ref[0] / denom


def kernel(logits, targets):
    B, N = targets.shape
    total = B * N
    sc_cols = (3 * N) // 4
    chunk_cols = 4096

    mesh = plsc.VectorSubcoreMesh(core_axis_name="c", subcore_axis_name="s")
    sc_count = functools.partial(
        pl.kernel,
        mesh=mesh,
        out_type=jax.ShapeDtypeStruct((_NW * _L,), jnp.float32),
        scratch_types=[
            pltpu.VMEM((8, chunk_cols), jnp.float32),
            pltpu.VMEM((8, chunk_cols), jnp.float32),
            pltpu.VMEM((_L,), jnp.float32),
            pltpu.SemaphoreType.DMA,
            pltpu.SemaphoreType.DMA,
        ],
    )(functools.partial(_sc_count_body, sc_cols=sc_cols, chunk_cols=chunk_cols))

    cnt = sc_count(targets)

    grid = 8
    lblk = N // (4 * grid)
    tblk = (N - sc_cols) // grid
    toff = sc_cols // tblk
    zs = pl.pallas_call(
        _tc_dense_body,
        grid=(grid,),
        in_specs=[
            pl.BlockSpec((B, lblk), lambda i: (0, i)),
            pl.BlockSpec((B, lblk), lambda i, g=grid: (0, i + g)),
            pl.BlockSpec((B, lblk), lambda i, g=grid: (0, i + 2 * g)),
            pl.BlockSpec((B, lblk), lambda i, g=grid: (0, i + 3 * g)),
            pl.BlockSpec((B, tblk), lambda i, o=toff: (0, i + o)),
        ],
        out_specs=pl.BlockSpec(memory_space=pltpu.SMEM),
        out_shape=jax.ShapeDtypeStruct((2,), jnp.float32),
        scratch_shapes=[pltpu.SMEM((2,), jnp.float32)],
    )(logits, logits, logits, logits, targets)

    out = pl.pallas_call(
        functools.partial(
            _finalize_body, total=float(total), num_top_k=float(_LIKELIHOOD_TOPK * B)
        ),
        in_specs=[
            pl.BlockSpec((_NW * _L,), lambda: (0,)),
            pl.BlockSpec(memory_space=pltpu.SMEM),
        ],
        out_specs=pl.BlockSpec(memory_space=pltpu.SMEM),
        out_shape=jax.ShapeDtypeStruct((1, 1), jnp.float32),
    )(cnt, zs)
    return out[0, 0]
